# baseline (speedup 1.0000x reference)
import functools

import jax
import jax.numpy as jnp
from jax import lax
from jax.experimental import pallas as pl
from jax.experimental.pallas import tpu as pltpu
from jax.experimental.pallas import tpu_sc as plsc

_ROWS = 1024
_COLS = 100000
_D = 32

_RB = 256
_CB = 2048
_NCB = pl.cdiv(_COLS, _CB)


def _argmax_body(x_ref, idx_ref, max_s, arg_s):
    j = pl.program_id(1)

    @pl.when(j == 0)
    def _init():
        max_s[...] = jnp.full((_RB,), -jnp.inf, jnp.float32)
        arg_s[...] = jnp.zeros((_RB,), jnp.int32)

    vals = x_ref[...]
    cols = j * _CB + lax.broadcasted_iota(jnp.int32, (_RB, _CB), 1)
    vals = jnp.where(cols < _COLS, vals, -jnp.inf)
    bmax = jnp.max(vals, axis=1)
    bidx = jnp.min(jnp.where(vals == bmax[:, None], cols, _COLS), axis=1)
    upd = bmax > max_s[...]
    arg_s[...] = jnp.where(upd, bidx, arg_s[...])
    max_s[...] = jnp.where(upd, bmax, max_s[...])

    @pl.when(j == _NCB - 1)
    def _out():
        d = lax.broadcasted_iota(jnp.int32, (_RB, _D), 1)
        idx_ref[...] = arg_s[...][:, None] * _D + d


_argmax_call = pl.pallas_call(
    _argmax_body,
    grid=(_ROWS // _RB, _NCB),
    in_specs=[pl.BlockSpec((_RB, _CB), lambda i, j: (i, j))],
    out_specs=pl.BlockSpec((_RB, _D), lambda i, j: (i, 0)),
    out_shape=jax.ShapeDtypeStruct((_ROWS, _D), jnp.int32),
    scratch_shapes=[
        pltpu.VMEM((_RB,), jnp.float32),
        pltpu.VMEM((_RB,), jnp.int32),
    ],
)

_info = plsc.get_sparse_core_info()
_NW = _info.num_cores * _info.num_subcores
_BPW = _ROWS // _NW
_EPW = _BPW * _D
_CHUNK = 128
_NDMA = _EPW // _CHUNK


def _gather_body(table_hbm, eidx_hbm, out_hbm, eidx_v, out_v, sem):
    wid = lax.axis_index("s") * _info.num_cores + lax.axis_index("c")
    pltpu.sync_copy(eidx_hbm.at[pl.ds(wid * _EPW, _EPW)], eidx_v)

    cps = [
        pltpu.async_copy(
            table_hbm.at[eidx_v.at[pl.ds(j * _CHUNK, _CHUNK)]],
            out_v.at[pl.ds(j * _CHUNK, _CHUNK)],
            sem,
        )
        for j in range(_NDMA)
    ]
    for cp in cps:
        cp.wait()
    pltpu.sync_copy(out_v, out_hbm.at[pl.ds(wid * _EPW, _EPW)])


_gather_call = functools.partial(
    pl.kernel,
    mesh=plsc.VectorSubcoreMesh(core_axis_name="c", subcore_axis_name="s"),
    out_type=jax.ShapeDtypeStruct((_ROWS * _D,), jnp.float32),
    scratch_types=[
        pltpu.VMEM((_EPW,), jnp.int32),
        pltpu.VMEM((_EPW,), jnp.float32),
        pltpu.SemaphoreType.DMA,
    ],
)(_gather_body)


def kernel(x, table):
    eidx = _argmax_call(x)
    flat = _gather_call(table.reshape(-1), eidx.reshape(-1))
    return flat.reshape(_ROWS, _D)
